# baseline (speedup 1.0000x reference)
import functools

import jax
import jax.numpy as jnp
from jax import lax
from jax.experimental import pallas as pl
from jax.experimental.pallas import tpu as pltpu
from jax.experimental.pallas import tpu_sc as plsc

N = 10000
NP = 10240
E = 320000
H = 128
G = 64
OUT = 64
CHUNK = 128
ECH = E // CHUNK
SB = 16
NCHP = 2528
NTILE = 32
RPT = NP // NTILE
BLK = 1024

_mesh = plsc.VectorSubcoreMesh(core_axis_name="c", subcore_axis_name="s")

_sc_params = pltpu.CompilerParams(needs_layout_passes=False)

_f32 = jnp.float32


def _agg_body(t_hbm, src_hbm, dst_hbm, meta_hbm, out_hbm,
              meta_v, sidx_v, didx_v, rows_v, acc_v,
              sem0, sem1, sem2, sem3):
    c = lax.axis_index("c")
    s = lax.axis_index("s")
    tid = c * 16 + s
    pltpu.sync_copy(meta_hbm, meta_v)
    cs = meta_v[tid, pl.ds(0, 16)][0]
    ce = meta_v[NTILE + tid, pl.ds(0, 16)][0]
    lo = tid * RPT

    zero16 = jnp.zeros((16,), _f32)

    def fillz(i, carry):
        for q in range(H // 16):
            acc_v[i, pl.ds(16 * q, 16)] = zero16
        return carry

    lax.fori_loop(0, RPT + 1, fillz, 0)

    sems = (sem0, sem1, sem2, sem3)
    NBUF = 4

    def scan_chunk(k, b, carry):
        def grp(j, gcarry):
            dvec = didx_v[k, pl.ds(j * 16, 16)]
            dl = dvec - lo
            inr = (dl >= 0) & (dl < RPT)
            rowv = jnp.where(inr, dl, RPT)
            for e in range(16):
                prev = gcarry[0]
                accs = list(gcarry[1:])
                d = dvec[e]
                row = rowv[e]
                same = d == prev
                ei = j * 16 + e
                for q in range(H // 16):
                    r = rows_v[b, ei, pl.ds(16 * q, 16)]
                    accs[q] = jnp.where(same, accs[q] + r, r)
                    acc_v[row, pl.ds(16 * q, 16)] = accs[q]
                gcarry = (d,) + tuple(accs)
            return gcarry

        return lax.fori_loop(0, CHUNK // 16, grp, carry)

    def super_body(sb, carry):
        c0 = pl.multiple_of(cs + sb * SB, 8)
        pltpu.sync_copy(src_hbm.at[pl.ds(c0, SB)], sidx_v)
        pltpu.sync_copy(dst_hbm.at[pl.ds(c0, SB)], didx_v)
        for b in range(NBUF):
            pltpu.make_async_copy(
                t_hbm.at[sidx_v.at[b]], rows_v.at[b], sems[b]).start()

        def chunk_quad(kk, ccarry):
            for b in range(NBUF):
                k = kk * NBUF + b
                pltpu.make_async_copy(
                    t_hbm.at[sidx_v.at[k]], rows_v.at[b], sems[b]).wait()
                ccarry = scan_chunk(k, b, ccarry)
                nk = k + NBUF

                @pl.when(nk < SB)
                def _():
                    pltpu.make_async_copy(
                        t_hbm.at[sidx_v.at[nk]], rows_v.at[b], sems[b]).start()
            return ccarry

        return lax.fori_loop(0, SB // NBUF, chunk_quad, carry)

    z16 = jnp.zeros((16,), _f32)
    carry0 = (jnp.int32(-1),) + (z16,) * (H // 16)
    nsuper = (ce - cs + SB - 1) // SB
    lax.fori_loop(0, nsuper, super_body, carry0)

    for k in range(RPT // 64):
        pltpu.sync_copy(acc_v.at[pl.ds(k * 64, 64)],
                        out_hbm.at[pl.ds(lo + k * 64, 64)])


_agg_call = functools.partial(
    pl.kernel,
    out_type=jax.ShapeDtypeStruct((NP, H), _f32),
    mesh=_mesh,
    scratch_types=[
        pltpu.VMEM((2 * NTILE, 16), jnp.int32),
        pltpu.VMEM((SB, CHUNK), jnp.int32),
        pltpu.VMEM((SB, CHUNK), jnp.int32),
        pltpu.VMEM((4, CHUNK, H), _f32),
        pltpu.VMEM((RPT + 1, H), _f32),
        pltpu.SemaphoreType.DMA,
        pltpu.SemaphoreType.DMA,
        pltpu.SemaphoreType.DMA,
        pltpu.SemaphoreType.DMA,
    ],
    compiler_params=_sc_params,
)(_agg_body)


def _deg_body(dst_hbm, meta_hbm, out_hbm, meta_v, didx_v, cnt_v):
    c = lax.axis_index("c")
    s = lax.axis_index("s")
    tid = c * 16 + s
    pltpu.sync_copy(meta_hbm, meta_v)
    cs = meta_v[tid, pl.ds(0, 16)][0]
    ce = meta_v[NTILE + tid, pl.ds(0, 16)][0]
    lo = tid * RPT

    zero16 = jnp.zeros((16,), _f32)
    one16 = jnp.ones((16,), _f32)

    def fillz(i, carry):
        cnt_v[i] = zero16
        return carry

    lax.fori_loop(0, RPT + 1, fillz, 0)

    def super_body(sb, carry):
        c0 = pl.multiple_of(cs + sb * SB, 8)
        pltpu.sync_copy(dst_hbm.at[pl.ds(c0, SB)], didx_v)

        def chunk(k, ccarry):
            def grp(j, gcarry):
                dvec = didx_v[k, pl.ds(j * 16, 16)]
                dl = dvec - lo
                inr = (dl >= 0) & (dl < RPT)
                rowv = jnp.where(inr, dl, RPT)
                for e in range(16):
                    prev, a = gcarry
                    d = dvec[e]
                    a = jnp.where(d == prev, a + one16, one16)
                    cnt_v[rowv[e]] = a
                    gcarry = (d, a)
                return gcarry

            return lax.fori_loop(0, CHUNK // 16, grp, ccarry)

        return lax.fori_loop(0, SB, chunk, carry)

    carry0 = (jnp.int32(-1), jnp.zeros((16,), _f32))
    nsuper = (ce - cs + SB - 1) // SB
    lax.fori_loop(0, nsuper, super_body, carry0)

    for k in range(RPT // 64):
        pltpu.sync_copy(cnt_v.at[pl.ds(k * 64, 64)],
                        out_hbm.at[pl.ds(lo + k * 64, 64)])


_deg_call = functools.partial(
    pl.kernel,
    out_type=jax.ShapeDtypeStruct((NP, 16), _f32),
    mesh=_mesh,
    scratch_types=[
        pltpu.VMEM((2 * NTILE, 16), jnp.int32),
        pltpu.VMEM((SB, CHUNK), jnp.int32),
        pltpu.VMEM((RPT + 1, 16), _f32),
    ],
    compiler_params=_sc_params,
)(_deg_body)


def _tc1_body(deg_ref, x_ref, w_ref, t_ref, dinv_ref):
    degs = deg_ref[:, 0:1] + 1.0
    dinv = lax.rsqrt(degs)
    dinvb = jnp.broadcast_to(dinv, (BLK, H))
    t_ref[...] = dinvb * jnp.dot(x_ref[...], w_ref[...],
                                 preferred_element_type=_f32)
    dinv_ref[...] = dinvb


def _tc1(degp, xp, W1):
    return pl.pallas_call(
        _tc1_body,
        grid=(NP // BLK,),
        in_specs=[
            pl.BlockSpec((BLK, 16), lambda i: (i, 0)),
            pl.BlockSpec((BLK, H), lambda i: (i, 0)),
            pl.BlockSpec((H, H), lambda i: (0, 0)),
        ],
        out_specs=[
            pl.BlockSpec((BLK, H), lambda i: (i, 0)),
            pl.BlockSpec((BLK, H), lambda i: (i, 0)),
        ],
        out_shape=[
            jax.ShapeDtypeStruct((NP, H), _f32),
            jax.ShapeDtypeStruct((NP, H), _f32),
        ],
    )(degp, xp, W1)


def _mid_body(u_ref, t_ref, dinv_ref, b_ref, w_ref, o_ref):
    h = jnp.maximum(dinv_ref[...] * (u_ref[...] + t_ref[...]) + b_ref[...],
                    0.0)
    o_ref[...] = dinv_ref[...] * jnp.dot(h, w_ref[...],
                                         preferred_element_type=_f32)


def _mid(u, t, dinvb, b, W):
    return pl.pallas_call(
        _mid_body,
        grid=(NP // BLK,),
        in_specs=[
            pl.BlockSpec((BLK, H), lambda i: (i, 0)),
            pl.BlockSpec((BLK, H), lambda i: (i, 0)),
            pl.BlockSpec((BLK, H), lambda i: (i, 0)),
            pl.BlockSpec((1, H), lambda i: (0, 0)),
            pl.BlockSpec((H, H), lambda i: (0, 0)),
        ],
        out_specs=pl.BlockSpec((BLK, H), lambda i: (i, 0)),
        out_shape=jax.ShapeDtypeStruct((NP, H), _f32),
    )(u, t, dinvb, b, W)


def _fin_body(u_ref, t_ref, dinv_ref, b_ref, batch_ref, wl1_ref, bl1_ref,
              wl2_ref, bl2_ref, o_ref):
    h = jnp.maximum(dinv_ref[...] * (u_ref[...] + t_ref[...]) + b_ref[...],
                    0.0)
    bm = batch_ref[...].reshape(1, NP)
    m = (lax.broadcasted_iota(jnp.int32, (G, NP), 0) == bm).astype(_f32)
    sums = jnp.dot(m, h, preferred_element_type=_f32)
    counts = jnp.sum(m, axis=1, keepdims=True)
    g = sums / jnp.maximum(counts, 1.0)
    z = jnp.maximum(
        jnp.dot(g, wl1_ref[...], preferred_element_type=_f32) + bl1_ref[...], 0.0)
    o_ref[...] = jnp.dot(z, wl2_ref[...], preferred_element_type=_f32) + bl2_ref[...]


def _final(u, t, dinvb, b, bp, Wl1, bl1, Wl2, bl2):
    return pl.pallas_call(
        _fin_body,
        out_shape=jax.ShapeDtypeStruct((G, OUT), _f32),
    )(u, t, dinvb, b, bp, Wl1, bl1, Wl2, bl2)


def kernel(x, edge_index, batch, W1, b1, W2, b2, W3, b3, W4, b4,
           Wl1, bl1, Wl2, bl2):
    src = edge_index[0]
    dst = edge_index[1]
    dst_s, src_s = lax.sort((dst, src), num_keys=1)
    padn = NCHP * CHUNK - E
    src2 = jnp.concatenate(
        [src_s, jnp.zeros((padn,), jnp.int32)]).reshape(NCHP, CHUNK)
    dst2 = jnp.concatenate(
        [dst_s, jnp.full((padn,), NP, jnp.int32)]).reshape(NCHP, CHUNK)
    bounds = jnp.arange(NTILE + 1, dtype=jnp.int32) * RPT
    pos = jnp.searchsorted(dst_s, bounds, side="left").astype(jnp.int32)
    cs = (pos[:NTILE] // CHUNK) // 8 * 8
    ce = (pos[1:] + CHUNK - 1) // CHUNK
    meta = jnp.tile(jnp.concatenate([cs, ce])[:, None], (1, 16))

    xp = jnp.pad(x, ((0, NP - N), (0, 0)))
    bp = jnp.pad(batch, (0, NP - N), constant_values=G)

    degp = _deg_call(dst2, meta)
    t1, dinvb = _tc1(degp, xp, W1)
    u1 = _agg_call(t1, src2, dst2, meta)
    t2 = _mid(u1, t1, dinvb, b1.reshape(1, H), W2)
    u2 = _agg_call(t2, src2, dst2, meta)
    t3 = _mid(u2, t2, dinvb, b2.reshape(1, H), W3)
    u3 = _agg_call(t3, src2, dst2, meta)
    t4 = _mid(u3, t3, dinvb, b3.reshape(1, H), W4)
    u4 = _agg_call(t4, src2, dst2, meta)
    return _final(u4, t4, dinvb, b4.reshape(1, H), bp,
                  Wl1, bl1.reshape(1, H), Wl2, bl2.reshape(1, OUT))


# speedup vs baseline: 1.3122x; 1.3122x over previous
import functools

import jax
import jax.numpy as jnp
from jax import lax
from jax.experimental import pallas as pl
from jax.experimental.pallas import tpu as pltpu
from jax.experimental.pallas import tpu_sc as plsc

N = 10000
NP = 10240
E = 320000
H = 128
G = 64
OUT = 64
CHUNK = 128
ECH = E // CHUNK
SB = 16
NCHP = 2528
NTILE = 32
RPT = NP // NTILE
BLK = 1024

_mesh = plsc.VectorSubcoreMesh(core_axis_name="c", subcore_axis_name="s")

_sc_params = pltpu.CompilerParams(needs_layout_passes=False)

_f32 = jnp.float32


def _agg_body(t_hbm, src_hbm, dst_hbm, meta_hbm, out_hbm,
              meta_v, sidx_v, didx_v, rows_v, acc_v, sem0, sem1):
    c = lax.axis_index("c")
    s = lax.axis_index("s")
    tid = c * 16 + s
    pltpu.sync_copy(meta_hbm, meta_v)
    cs = meta_v[tid, pl.ds(0, 16)][0]
    ce = meta_v[NTILE + tid, pl.ds(0, 16)][0]
    lo = tid * RPT

    zero16 = jnp.zeros((16,), _f32)

    def fillz(i, carry):
        for q in range(H // 16):
            acc_v[i, pl.ds(16 * q, 16)] = zero16
        return carry

    lax.fori_loop(0, RPT + 1, fillz, 0)

    sems = (sem0, sem1)
    cols = [jnp.arange(16, dtype=jnp.int32) + 16 * q for q in range(H // 16)]

    def scan_chunk(k, b):
        def grp(j, gcarry):
            dvec = didx_v[k, pl.ds(j * 16, 16)]
            dl = dvec - lo
            inr = (dl >= 0) & (dl < RPT)
            rowv = jnp.where(inr, dl, RPT)
            for e in range(16):
                row16 = jnp.broadcast_to(rowv[e], (16,))
                ei = j * 16 + e
                for q in range(H // 16):
                    r = rows_v[b, ei, pl.ds(16 * q, 16)]
                    plsc.addupdate_scatter(acc_v, [row16, cols[q]], r)
            return gcarry

        lax.fori_loop(0, CHUNK // 16, grp, 0)

    def super_body(sb, carry):
        c0 = pl.multiple_of(cs + sb * SB, 8)
        pltpu.sync_copy(src_hbm.at[pl.ds(c0, SB)], sidx_v)
        pltpu.sync_copy(dst_hbm.at[pl.ds(c0, SB)], didx_v)
        for b in range(2):
            pltpu.make_async_copy(
                t_hbm.at[sidx_v.at[b]], rows_v.at[b], sems[b]).start()

        def chunk_pair(kk, ccarry):
            for b in range(2):
                k = kk * 2 + b
                pltpu.make_async_copy(
                    t_hbm.at[sidx_v.at[k]], rows_v.at[b], sems[b]).wait()
                scan_chunk(k, b)
                nk = k + 2

                @pl.when(nk < SB)
                def _():
                    pltpu.make_async_copy(
                        t_hbm.at[sidx_v.at[nk]], rows_v.at[b], sems[b]).start()
            return ccarry

        return lax.fori_loop(0, SB // 2, chunk_pair, carry)

    nsuper = (ce - cs + SB - 1) // SB
    lax.fori_loop(0, nsuper, super_body, 0)

    for k in range(RPT // 64):
        pltpu.sync_copy(acc_v.at[pl.ds(k * 64, 64)],
                        out_hbm.at[pl.ds(lo + k * 64, 64)])


_agg_call = functools.partial(
    pl.kernel,
    out_type=jax.ShapeDtypeStruct((NP, H), _f32),
    mesh=_mesh,
    scratch_types=[
        pltpu.VMEM((2 * NTILE, 16), jnp.int32),
        pltpu.VMEM((SB, CHUNK), jnp.int32),
        pltpu.VMEM((SB, CHUNK), jnp.int32),
        pltpu.VMEM((2, CHUNK, H), _f32),
        pltpu.VMEM((RPT + 1, H), _f32),
        pltpu.SemaphoreType.DMA,
        pltpu.SemaphoreType.DMA,
    ],
    compiler_params=_sc_params,
)(_agg_body)


def _deg_body(dst_hbm, meta_hbm, out_hbm, meta_v, didx_v, cnt_v):
    c = lax.axis_index("c")
    s = lax.axis_index("s")
    tid = c * 16 + s
    pltpu.sync_copy(meta_hbm, meta_v)
    cs = meta_v[tid, pl.ds(0, 16)][0]
    ce = meta_v[NTILE + tid, pl.ds(0, 16)][0]
    lo = tid * RPT

    zero16 = jnp.zeros((16,), _f32)
    one16 = jnp.ones((16,), _f32)

    def fillz(i, carry):
        cnt_v[i] = zero16
        return carry

    lax.fori_loop(0, RPT + 1, fillz, 0)

    def super_body(sb, carry):
        c0 = pl.multiple_of(cs + sb * SB, 8)
        pltpu.sync_copy(dst_hbm.at[pl.ds(c0, SB)], didx_v)

        def chunk(k, ccarry):
            def grp(j, gcarry):
                dvec = didx_v[k, pl.ds(j * 16, 16)]
                dl = dvec - lo
                inr = (dl >= 0) & (dl < RPT)
                rowv = jnp.where(inr, dl, RPT)
                for e in range(16):
                    prev, a = gcarry
                    d = dvec[e]
                    a = jnp.where(d == prev, a + one16, one16)
                    cnt_v[rowv[e]] = a
                    gcarry = (d, a)
                return gcarry

            return lax.fori_loop(0, CHUNK // 16, grp, ccarry)

        return lax.fori_loop(0, SB, chunk, carry)

    carry0 = (jnp.int32(-1), jnp.zeros((16,), _f32))
    nsuper = (ce - cs + SB - 1) // SB
    lax.fori_loop(0, nsuper, super_body, carry0)

    for k in range(RPT // 64):
        pltpu.sync_copy(cnt_v.at[pl.ds(k * 64, 64)],
                        out_hbm.at[pl.ds(lo + k * 64, 64)])


_deg_call = functools.partial(
    pl.kernel,
    out_type=jax.ShapeDtypeStruct((NP, 16), _f32),
    mesh=_mesh,
    scratch_types=[
        pltpu.VMEM((2 * NTILE, 16), jnp.int32),
        pltpu.VMEM((SB, CHUNK), jnp.int32),
        pltpu.VMEM((RPT + 1, 16), _f32),
    ],
    compiler_params=_sc_params,
)(_deg_body)


def _tc1_body(deg_ref, x_ref, w_ref, t_ref, dinv_ref):
    degs = deg_ref[:, 0:1] + 1.0
    dinv = lax.rsqrt(degs)
    dinvb = jnp.broadcast_to(dinv, (BLK, H))
    t_ref[...] = dinvb * jnp.dot(x_ref[...], w_ref[...],
                                 preferred_element_type=_f32)
    dinv_ref[...] = dinvb


def _tc1(degp, xp, W1):
    return pl.pallas_call(
        _tc1_body,
        grid=(NP // BLK,),
        in_specs=[
            pl.BlockSpec((BLK, 16), lambda i: (i, 0)),
            pl.BlockSpec((BLK, H), lambda i: (i, 0)),
            pl.BlockSpec((H, H), lambda i: (0, 0)),
        ],
        out_specs=[
            pl.BlockSpec((BLK, H), lambda i: (i, 0)),
            pl.BlockSpec((BLK, H), lambda i: (i, 0)),
        ],
        out_shape=[
            jax.ShapeDtypeStruct((NP, H), _f32),
            jax.ShapeDtypeStruct((NP, H), _f32),
        ],
    )(degp, xp, W1)


def _mid_body(u_ref, t_ref, dinv_ref, b_ref, w_ref, o_ref):
    h = jnp.maximum(dinv_ref[...] * (u_ref[...] + t_ref[...]) + b_ref[...],
                    0.0)
    o_ref[...] = dinv_ref[...] * jnp.dot(h, w_ref[...],
                                         preferred_element_type=_f32)


def _mid(u, t, dinvb, b, W):
    return pl.pallas_call(
        _mid_body,
        grid=(NP // BLK,),
        in_specs=[
            pl.BlockSpec((BLK, H), lambda i: (i, 0)),
            pl.BlockSpec((BLK, H), lambda i: (i, 0)),
            pl.BlockSpec((BLK, H), lambda i: (i, 0)),
            pl.BlockSpec((1, H), lambda i: (0, 0)),
            pl.BlockSpec((H, H), lambda i: (0, 0)),
        ],
        out_specs=pl.BlockSpec((BLK, H), lambda i: (i, 0)),
        out_shape=jax.ShapeDtypeStruct((NP, H), _f32),
    )(u, t, dinvb, b, W)


def _fin_body(u_ref, t_ref, dinv_ref, b_ref, batch_ref, wl1_ref, bl1_ref,
              wl2_ref, bl2_ref, o_ref):
    h = jnp.maximum(dinv_ref[...] * (u_ref[...] + t_ref[...]) + b_ref[...],
                    0.0)
    bm = batch_ref[...].reshape(1, NP)
    m = (lax.broadcasted_iota(jnp.int32, (G, NP), 0) == bm).astype(_f32)
    sums = jnp.dot(m, h, preferred_element_type=_f32)
    counts = jnp.sum(m, axis=1, keepdims=True)
    g = sums / jnp.maximum(counts, 1.0)
    z = jnp.maximum(
        jnp.dot(g, wl1_ref[...], preferred_element_type=_f32) + bl1_ref[...], 0.0)
    o_ref[...] = jnp.dot(z, wl2_ref[...], preferred_element_type=_f32) + bl2_ref[...]


def _final(u, t, dinvb, b, bp, Wl1, bl1, Wl2, bl2):
    return pl.pallas_call(
        _fin_body,
        out_shape=jax.ShapeDtypeStruct((G, OUT), _f32),
    )(u, t, dinvb, b, bp, Wl1, bl1, Wl2, bl2)


def kernel(x, edge_index, batch, W1, b1, W2, b2, W3, b3, W4, b4,
           Wl1, bl1, Wl2, bl2):
    src = edge_index[0]
    dst = edge_index[1]
    dst_s, src_s = lax.sort((dst, src), num_keys=1)
    padn = NCHP * CHUNK - E
    src2 = jnp.concatenate(
        [src_s, jnp.zeros((padn,), jnp.int32)]).reshape(NCHP, CHUNK)
    dst2 = jnp.concatenate(
        [dst_s, jnp.full((padn,), NP, jnp.int32)]).reshape(NCHP, CHUNK)
    bounds = jnp.arange(NTILE + 1, dtype=jnp.int32) * RPT
    pos = jnp.searchsorted(dst_s, bounds, side="left").astype(jnp.int32)
    cs = (pos[:NTILE] // CHUNK) // 8 * 8
    ce = (pos[1:] + CHUNK - 1) // CHUNK
    meta = jnp.tile(jnp.concatenate([cs, ce])[:, None], (1, 16))

    xp = jnp.pad(x, ((0, NP - N), (0, 0)))
    bp = jnp.pad(batch, (0, NP - N), constant_values=G)

    degp = _deg_call(dst2, meta)
    t1, dinvb = _tc1(degp, xp, W1)
    u1 = _agg_call(t1, src2, dst2, meta)
    t2 = _mid(u1, t1, dinvb, b1.reshape(1, H), W2)
    u2 = _agg_call(t2, src2, dst2, meta)
    t3 = _mid(u2, t2, dinvb, b2.reshape(1, H), W3)
    u3 = _agg_call(t3, src2, dst2, meta)
    t4 = _mid(u3, t3, dinvb, b3.reshape(1, H), W4)
    u4 = _agg_call(t4, src2, dst2, meta)
    return _final(u4, t4, dinvb, b4.reshape(1, H), bp,
                  Wl1, bl1.reshape(1, H), Wl2, bl2.reshape(1, OUT))
